# Initial kernel scaffold; baseline (speedup 1.0000x reference)
import jax
import jax.numpy as jnp
from jax.experimental import pallas as pl


def kernel(inputs, emb):
    raise NotImplementedError("write your pallas kernel here")


# speedup vs baseline: 6.4689x; 6.4689x over previous
import functools

import jax
import jax.numpy as jnp
from jax import lax
from jax.experimental import pallas as pl
from jax.experimental.pallas import tpu as pltpu
from jax.experimental.pallas import tpu_sc as plsc

_NUM_EMB = 1024
_NUM_HEADS = 4
_EMB_DIM = 768
_HEAD_DIM = _EMB_DIM // _NUM_HEADS
_COMMIT = 0.25

_BN = 1024
_NW = 32


_KA = 256


def _tc_body(x_ref, w_ref, codes_ref, codesoff_ref, lsum_ref,
             waug_ref, xaug_ref):
    @pl.when(pl.program_id(0) == 0)
    def _init():
        lane = lax.broadcasted_iota(jnp.int32, (_HEAD_DIM, _KA - _HEAD_DIM), 1)
        nh = jnp.where(lane == 0, jnp.float32(-0.5), 0.0).astype(jnp.bfloat16)
        dn = (((1,), (0,)), ((), ()))
        for h in range(_NUM_HEADS):
            w = w_ref[h]
            w2 = w * w
            a1 = w2.astype(jnp.bfloat16)
            r1 = w2 - a1.astype(jnp.float32)
            a2 = r1.astype(jnp.bfloat16)
            a3 = (r1 - a2.astype(jnp.float32)).astype(jnp.bfloat16)
            acc = lax.dot_general(a1, nh, dn,
                                  preferred_element_type=jnp.float32)
            acc += lax.dot_general(a2, nh, dn,
                                   preferred_element_type=jnp.float32)
            acc += lax.dot_general(a3, nh, dn,
                                   preferred_element_type=jnp.float32)
            b1 = acc.astype(jnp.bfloat16).astype(jnp.float32)
            r = acc - b1
            b2 = r.astype(jnp.bfloat16).astype(jnp.float32)
            b3 = (r - b2).astype(jnp.bfloat16).astype(jnp.float32)
            bias = b1 + jnp.roll(b2, 1, axis=1) + jnp.roll(b3, 2, axis=1)
            waug_ref[h, :, :_HEAD_DIM] = w
            waug_ref[h, :, _HEAD_DIM:] = bias
        lane2 = lax.broadcasted_iota(jnp.int32, (_BN, _KA - _HEAD_DIM), 1)
        xaug_ref[:, _HEAD_DIM:] = jnp.where(lane2 < 3, jnp.float32(1.0), 0.0)
        lsum_ref[0, :] = jnp.zeros((128,), jnp.float32)

    xall = x_ref[...]
    x2s = lax.dot_general(
        jnp.ones((8, _BN), jnp.float32), xall * xall,
        (((1,), (0,)), ((), ())), preferred_element_type=jnp.float32)
    part = jnp.sum(x2s[0, :])
    for h in range(_NUM_HEADS):
        xaug_ref[:, :_HEAD_DIM] = x_ref[:, h * _HEAD_DIM:(h + 1) * _HEAD_DIM]
        eT = lax.dot_general(
            waug_ref[h], xaug_ref[...], (((1,), (1,)), ((), ())),
            preferred_element_type=jnp.float32)
        m = jnp.max(eT, axis=0)
        idx = jnp.argmax(eT, axis=0).astype(jnp.int32)
        part += -2.0 * jnp.sum(m)
        codes_ref[h, 0, :] = idx
        codesoff_ref[h, 0, :] = idx + h * _NUM_EMB

    lsum_ref[0, :] += jnp.full((128,), part, jnp.float32)


def _tc_distance_argmax(x2, emb, row0, nrows):
    b0 = row0 // _BN
    nb = nrows // _BN
    return pl.pallas_call(
        _tc_body,
        grid=(nb,),
        in_specs=[
            pl.BlockSpec((_BN, _EMB_DIM), lambda i: (b0 + i, 0)),
            pl.BlockSpec((_NUM_HEADS, _NUM_EMB, _HEAD_DIM), lambda i: (0, 0, 0)),
        ],
        out_specs=[
            pl.BlockSpec((_NUM_HEADS, 1, _BN), lambda i: (0, 0, i)),
            pl.BlockSpec((_NUM_HEADS, 1, _BN), lambda i: (0, 0, i)),
            pl.BlockSpec((1, 128), lambda i: (0, 0)),
        ],
        out_shape=[
            jax.ShapeDtypeStruct((_NUM_HEADS, 1, nrows), jnp.int32),
            jax.ShapeDtypeStruct((_NUM_HEADS, 1, nrows), jnp.int32),
            jax.ShapeDtypeStruct((1, 128), jnp.float32),
        ],
        scratch_shapes=[
            pltpu.VMEM((_NUM_HEADS, _NUM_EMB, _KA), jnp.float32),
            pltpu.VMEM((_BN, _KA), jnp.float32),
        ],
    )(x2, emb)


def _sc_gather(table, idx3, n):
    hh, kw, b_per_w = idx3.shape
    d = table.shape[1]
    ch = next(c for c in (128, 96, 64, 32) if b_per_w % c == 0)
    nc_chunks = b_per_w // ch
    mesh = plsc.VectorSubcoreMesh(core_axis_name="c", subcore_axis_name="s")

    @functools.partial(
        pl.kernel,
        mesh=mesh,
        out_type=jax.ShapeDtypeStruct((n, 1, hh * d), jnp.float32),
        compiler_params=pltpu.CompilerParams(use_tc_tiling_on_sc=False),
        scratch_types=[
            pltpu.VMEM_SHARED(table.shape, jnp.float32),
            pltpu.VMEM((b_per_w,), jnp.int32),
            pltpu.VMEM((ch, d), jnp.float32),
            pltpu.VMEM((ch, d), jnp.float32),
            pltpu.VMEM((ch, d), jnp.float32),
            pltpu.SemaphoreType.DMA,
            pltpu.SemaphoreType.DMA,
            pltpu.SemaphoreType.DMA,
            pltpu.SemaphoreType.DMA,
            pltpu.SemaphoreType.DMA,
            pltpu.SemaphoreType.DMA,
        ],
    )
    def k(table_hbm, idx_hbm, out_hbm, tbl_s, idx_v,
          b0, b1, b2, g0, g1, g2, w0, w1, w2):
        cid = lax.axis_index("c")
        sid = lax.axis_index("s")
        wid = sid * 2 + cid
        h = wid // (32 // hh)
        kk = wid % (32 // hh)
        bufs = (b0, b1, b2)
        gsems = (g0, g1, g2)
        wsems = (w0, w1, w2)

        vslice = table.shape[0] // 16
        pltpu.sync_copy(table_hbm.at[pl.ds(sid * vslice, vslice)],
                        tbl_s.at[pl.ds(sid * vslice, vslice)])
        plsc.subcore_barrier()
        pltpu.sync_copy(idx_hbm.at[h, kk], idx_v)

        def gather(j):
            b = j % 3
            return pltpu.async_copy(
                tbl_s.at[idx_v.at[pl.ds(j * ch, ch)]], bufs[b], gsems[b])

        def write(j):
            b = j % 3
            return pltpu.async_copy(
                bufs[b],
                out_hbm.at[pl.ds(kk * b_per_w + j * ch, ch), 0,
                           pl.ds(h * d, d)],
                wsems[b])

        ghs = {0: gather(0), 1: gather(1)}
        whs = {}
        for j in range(nc_chunks):
            ghs[j].wait()
            whs[j] = write(j)
            nxt = j + 2
            if nxt < nc_chunks:
                if nxt - 3 >= 0:
                    whs[nxt - 3].wait()
                ghs[nxt] = gather(nxt)
        for j in range(max(0, nc_chunks - 3), nc_chunks):
            whs[j].wait()

    return k(table, idx3)


_G = 1


def kernel(inputs, emb):
    n = inputs.shape[0]
    x2 = inputs.reshape(n, _EMB_DIM)
    table = emb.reshape(_NUM_HEADS * _NUM_EMB, _HEAD_DIM)
    kw = _NW // _NUM_HEADS
    ng = n // _G

    q_parts, codes_parts, lsums = [], [], []
    for g in range(_G):
        codes4, codesoff4, lsum = _tc_distance_argmax(x2, emb, g * ng, ng)
        idx3 = codesoff4.reshape(_NUM_HEADS, kw, ng // kw)
        q = _sc_gather(table, idx3, ng)
        q_parts.append(q)
        codes_parts.append(codes4.reshape(_NUM_HEADS, ng).T)
        lsums.append(lsum[0, 0])

    if _G == 1:
        quantized_st, vq_codes = q_parts[0], codes_parts[0]
    else:
        quantized_st = jnp.concatenate(q_parts, axis=0)
        vq_codes = jnp.concatenate(codes_parts, axis=0)
    loss = _COMMIT * sum(lsums) / (n * _EMB_DIM)
    return quantized_st, loss, vq_codes
